# baseline (speedup 1.0000x reference)
import functools

import jax
import jax.numpy as jnp
from jax import lax
from jax.experimental import pallas as pl
from jax.experimental.pallas import tpu as pltpu
from jax.experimental.pallas import tpu_sc as plsc

_NBUF = 3


def _make_gather(b: int, l: int, vocab: int, d: int):
    info = plsc.get_sparse_core_info()
    nc, ns = info.num_cores, info.num_subcores
    nw = nc * ns
    lanes = info.num_lanes

    bt_n, bl_n = nw, b // nw
    ct_n, cs_n = d // 8, 8
    lh = 4
    n_k = l // lh
    chunk = lh * bl_n
    pitch = bl_n + 1
    assert bl_n == 128 and d == 32 and l % lh == 0 and n_k % _NBUF != 0

    mesh = plsc.VectorSubcoreMesh(core_axis_name="c", subcore_axis_name="s")

    @functools.partial(
        pl.kernel,
        mesh=mesh,
        compiler_params=pltpu.CompilerParams(
            use_tc_tiling_on_sc=False, needs_layout_passes=False
        ),
        out_type=jax.ShapeDtypeStruct((l, ct_n, bt_n, cs_n, bl_n), jnp.float32),
        scratch_types=[
            pltpu.VMEM((bl_n, l), jnp.int32),
        ]
        + [pltpu.VMEM((chunk,), jnp.int32)] * _NBUF
        + [pltpu.VMEM((chunk, d), jnp.float32)] * _NBUF
        + [pltpu.VMEM((lh, ct_n, cs_n, pitch), jnp.float32)]
        + [pltpu.SemaphoreType.DMA] * (_NBUF + 1),
    )
    def gather_kernel(idx_hbm, table_hbm, z_hbm, idx_vm, *scr):
        idx_f = scr[:_NBUF]
        rows = scr[_NBUF : 2 * _NBUF]
        zbuf = scr[2 * _NBUF]
        sem_g = scr[2 * _NBUF + 1 : 3 * _NBUF + 1]
        so = scr[3 * _NBUF + 1]
        w = lax.axis_index("s") * nc + lax.axis_index("c")
        pltpu.sync_copy(idx_hbm.at[pl.ds(w * bl_n, bl_n), :], idx_vm)
        lane = lax.iota(jnp.int32, lanes)
        ct_v = [jnp.right_shift(16 * h + lane, 3) for h in range(2)]
        cs_v = [jnp.bitwise_and(16 * h + lane, 7) for h in range(2)]

        def issue(k, p):
            def one(m, c2):
                e = m * lanes + lane
                vals = plsc.load_gather(
                    idx_vm,
                    [jnp.right_shift(e, 2), lh * k + jnp.bitwise_and(e, 3)],
                )
                idx_f[p][pl.ds(m * lanes, lanes)] = vals
                return c2

            lax.fori_loop(0, chunk // lanes, one, 0)
            pltpu.make_async_copy(table_hbm.at[idx_f[p]], rows[p], sem_g[p]).start()

        def gather_wait(p):
            pltpu.make_async_copy(table_hbm.at[idx_f[p]], rows[p], sem_g[p]).wait()

        def out_copy(k):
            return pltpu.make_async_copy(
                zbuf.at[:, :, :, pl.ds(0, bl_n)],
                z_hbm.at[pl.ds(k * lh, lh), :, w],
                so,
            )

        def shuffle(p):
            def grp_body(g, c2):
                ls = jnp.right_shift(g, 3)
                blg = jnp.bitwise_and(g, 7)
                ls_s = jnp.broadcast_to(ls, (lanes,))
                for bl_i in range(lanes):
                    bl = blg * lanes + bl_i
                    r = lh * bl + ls
                    bl_s = jnp.broadcast_to(bl, (lanes,))
                    for h in range(2):
                        vals = rows[p][r, pl.ds(h * lanes, lanes)]
                        plsc.store_scatter(
                            zbuf, [ls_s, ct_v[h], cs_v[h], bl_s], vals
                        )
                return c2

            lax.fori_loop(0, lh * 8, grp_body, 0)

        for k in range(_NBUF):
            issue(k, k)

        def tri_body(g, carry):
            for t in range(_NBUF):
                k = _NBUF * g + t
                gather_wait(t)
                lax.cond(k > 0, lambda: out_copy(k - 1).wait(), lambda: None)
                shuffle(t)
                out_copy(k).start()
                issue(k + _NBUF, t)
            return carry

        n_steady = n_k - _NBUF - (n_k % _NBUF)
        lax.fori_loop(0, n_steady // _NBUF, tri_body, 0)

        for k in range(n_steady, n_k):
            p = k % _NBUF
            gather_wait(p)
            out_copy(k - 1).wait()
            shuffle(p)
            out_copy(k).start()
            if k + _NBUF < n_k:
                issue(k + _NBUF, p)
        out_copy(n_k - 1).wait()

    return gather_kernel


def kernel(indices, table):
    b, l = indices.shape
    vocab, d = table.shape
    z = _make_gather(b, l, vocab, d)(indices, table)
    return z.transpose((2, 4, 0, 1, 3)).reshape(b, l, d)
